# baseline (speedup 1.0000x reference)
import functools

import jax
import jax.numpy as jnp
from jax import lax
from jax.experimental import pallas as pl
from jax.experimental.pallas import tpu as pltpu
from jax.experimental.pallas import tpu_sc as plsc

F32 = jnp.float32
H = 32
HH = 16
NC = 2
NS = 16
CH = 128
KCH_A = 5
KCH_D = 25
KCH_E = 4
CH_E = 128


def _axt_kernel(n, e, src_hbm, dst_hbm, tbl0, tbl1, zeros_hbm, out_hbm,
                acc, idxs0, idxd0, idxs1, idxd1, rows0, rows1,
                isem, gsem, ssem):
    c = lax.axis_index("c")
    s = lax.axis_index("s")
    rpt = n // NS
    nbt = e // CH // KCH_A
    nblk = (nbt - s + NS - 1) // NS
    pltpu.sync_copy(zeros_hbm.at[pl.ds(s * rpt, rpt)], acc.at[pl.ds(s * rpt, rpt)])
    pltpu.async_copy(src_hbm.at[pl.ds(s * KCH_A, KCH_A)], idxs0, isem)
    pltpu.async_copy(dst_hbm.at[pl.ds(s * KCH_A, KCH_A)], idxd0, isem)
    plsc.subcore_barrier()

    def fire_gathers(idxs, rows):
        @pl.when(c == 0)
        def _():
            for j in range(KCH_A):
                pltpu.async_copy(tbl0.at[idxs.at[j]], rows.at[j], gsem)

        @pl.when(c == 1)
        def _():
            for j in range(KCH_A):
                pltpu.async_copy(tbl1.at[idxs.at[j]], rows.at[j], gsem)

    def wait_gathers(idxs, rows):
        @pl.when(c == 0)
        def _():
            for j in range(KCH_A):
                pltpu.make_async_copy(tbl0.at[idxs.at[j]], rows.at[j],
                                      gsem).wait()

        @pl.when(c == 1)
        def _():
            for j in range(KCH_A):
                pltpu.make_async_copy(tbl1.at[idxs.at[j]], rows.at[j],
                                      gsem).wait()

    pltpu.make_async_copy(src_hbm.at[pl.ds(0, KCH_A)], idxs0, isem).wait()
    pltpu.make_async_copy(dst_hbm.at[pl.ds(0, KCH_A)], idxd0, isem).wait()

    @pl.when(1 < nblk)
    def _():
        row1 = (s + NS) * KCH_A
        pltpu.async_copy(src_hbm.at[pl.ds(row1, KCH_A)], idxs1, isem)
        pltpu.async_copy(dst_hbm.at[pl.ds(row1, KCH_A)], idxd1, isem)

    fire_gathers(idxs0, rows0)

    def blk_body(b, carry):
        def run(idxs, idxd, rows, oidxs, oidxd, orows):
            wait_gathers(idxs, rows)

            @pl.when(b + 1 < nblk)
            def _():
                pltpu.make_async_copy(src_hbm.at[pl.ds(0, KCH_A)], oidxs,
                                      isem).wait()
                pltpu.make_async_copy(dst_hbm.at[pl.ds(0, KCH_A)], oidxd,
                                      isem).wait()

            sds = [pltpu.async_copy(rows.at[j], acc.at[idxd.at[j]], ssem,
                                    add=True)
                   for j in range(KCH_A)]
            for d in sds:
                d.wait()

            @pl.when(b + 1 < nblk)
            def _():
                fire_gathers(oidxs, orows)

            @pl.when(b + 2 < nblk)
            def _():
                row2 = (s + (b + 2) * NS) * KCH_A
                pltpu.async_copy(src_hbm.at[pl.ds(row2, KCH_A)], idxs, isem)
                pltpu.async_copy(dst_hbm.at[pl.ds(row2, KCH_A)], idxd, isem)

        @pl.when(b % 2 == 0)
        def _():
            run(idxs0, idxd0, rows0, idxs1, idxd1, rows1)

        @pl.when(b % 2 == 1)
        def _():
            run(idxs1, idxd1, rows1, idxs0, idxd0, rows0)

        return carry

    lax.fori_loop(0, nblk, blk_body, 0)
def _make_axt(n, e):
    mesh = plsc.VectorSubcoreMesh(core_axis_name="c", subcore_axis_name="s")
    return functools.partial(
        pl.kernel, functools.partial(_axt_kernel, n, e),
        out_type=jax.ShapeDtypeStruct((NC, n, HH), F32),
        mesh=mesh,
        compiler_params=pltpu.CompilerParams(use_tc_tiling_on_sc=False),
        scratch_types=[
            pltpu.VMEM_SHARED((n, HH), F32),
            pltpu.VMEM((KCH_A, CH), jnp.int32),
            pltpu.VMEM((KCH_A, CH), jnp.int32),
            pltpu.VMEM((KCH_A, CH), jnp.int32),
            pltpu.VMEM((KCH_A, CH), jnp.int32),
            pltpu.VMEM((KCH_A, CH, HH), F32),
            pltpu.VMEM((KCH_A, CH, HH), F32),
            pltpu.SemaphoreType.DMA,
            pltpu.SemaphoreType.DMA,
            pltpu.SemaphoreType.DMA,
        ])()


def _deg_kernel(n, e, dst_hbm, ones_hbm, zeros_hbm, out_hbm,
                acc, idxd, ones_v, ssem):
    c = lax.axis_index("c")
    s = lax.axis_index("s")
    wid = c * NS + s
    rpt = n // NS
    nbt = e // CH // KCH_D
    nblk = (nbt - wid + NC * NS - 1) // (NC * NS)
    pltpu.sync_copy(zeros_hbm.at[pl.ds(s * rpt, rpt)], acc.at[pl.ds(s * rpt, rpt)])
    pltpu.sync_copy(ones_hbm, ones_v)
    plsc.subcore_barrier()

    def blk_body(b, carry):
        row0 = (wid + b * NC * NS) * KCH_D
        pltpu.sync_copy(dst_hbm.at[pl.ds(row0, KCH_D)], idxd)
        sds = [pltpu.async_copy(ones_v, acc.at[idxd.at[j]], ssem, add=True)
               for j in range(KCH_D)]
        for d in sds:
            d.wait()
        return carry

    lax.fori_loop(0, nblk, blk_body, 0)
    plsc.subcore_barrier()
    pltpu.sync_copy(acc.at[pl.ds(s * rpt, rpt)],
                    out_hbm.at[c, pl.ds(s * rpt, rpt)])


def _make_deg(n, e):
    mesh = plsc.VectorSubcoreMesh(core_axis_name="c", subcore_axis_name="s")
    return functools.partial(
        pl.kernel, functools.partial(_deg_kernel, n, e),
        out_type=jax.ShapeDtypeStruct((NC, n, HH), F32),
        mesh=mesh,
        compiler_params=pltpu.CompilerParams(use_tc_tiling_on_sc=False),
        scratch_types=[
            pltpu.VMEM_SHARED((n, HH), F32),
            pltpu.VMEM((KCH_D, CH), jnp.int32),
            pltpu.VMEM((CH, HH), F32),
            pltpu.SemaphoreType.DMA,
        ])()


def _edge_kernel(n, e, src_hbm, dst_hbm, w0_hbm, w1_hbm, u0, u1, v0, v1,
                 zeros_hbm, out_hbm, acc, idxs0, idxd0, idxs1, idxd1,
                 ub, vb, wb, isem, wsem, gsem1, gsem2, ssem):
    c = lax.axis_index("c")
    s = lax.axis_index("s")
    rpt = n // NS
    blk = CH_E * KCH_E
    nbt = e // blk
    nblk = (nbt - s + NS - 1) // NS
    pltpu.sync_copy(zeros_hbm.at[pl.ds(s * rpt, rpt)], acc.at[pl.ds(s * rpt, rpt)])
    pltpu.async_copy(src_hbm.at[pl.ds(s * KCH_E, KCH_E)], idxs0, isem)
    pltpu.async_copy(dst_hbm.at[pl.ds(s * KCH_E, KCH_E)], idxd0, isem)
    plsc.subcore_barrier()

    def blk_body(b, carry):
        def run(idxs, idxd, oidxs, oidxd):
            g = s + b * NS
            base = g * blk
            pltpu.make_async_copy(src_hbm.at[pl.ds(0, KCH_E)], idxs, isem).wait()
            pltpu.make_async_copy(dst_hbm.at[pl.ds(0, KCH_E)], idxd, isem).wait()

            @pl.when(b + 1 < nblk)
            def _():
                row1 = (s + (b + 1) * NS) * KCH_E
                pltpu.async_copy(src_hbm.at[pl.ds(row1, KCH_E)], oidxs, isem)
                pltpu.async_copy(dst_hbm.at[pl.ds(row1, KCH_E)], oidxd, isem)

            @pl.when(c == 0)
            def _():
                wd = pltpu.async_copy(w0_hbm.at[pl.ds(base // 8, blk // 8)],
                                      wb, wsem)
                gds = [pltpu.async_copy(u0.at[idxs.at[j]],
                                        ub.at[pl.ds(j * CH_E, CH_E)], gsem1)
                       for j in range(KCH_E)]
                gds += [pltpu.async_copy(v0.at[idxd.at[j]],
                                         vb.at[pl.ds(j * CH_E, CH_E)], gsem2)
                        for j in range(KCH_E)]
                for d in gds:
                    d.wait()
                wd.wait()

            @pl.when(c == 1)
            def _():
                wd = pltpu.async_copy(w1_hbm.at[pl.ds(base // 8, blk // 8)],
                                      wb, wsem)
                gds = [pltpu.async_copy(u1.at[idxs.at[j]],
                                        ub.at[pl.ds(j * CH_E, CH_E)], gsem1)
                       for j in range(KCH_E)]
                gds += [pltpu.async_copy(v1.at[idxd.at[j]],
                                         vb.at[pl.ds(j * CH_E, CH_E)], gsem2)
                        for j in range(KCH_E)]
                for d in gds:
                    d.wait()
                wd.wait()

            def oct_body(q, carry2):
                for kk in range(8):
                    r = q * 8 + kk
                    ub[r] = jnp.maximum(
                        ub[r] + vb[r] + wb[q, kk * HH:(kk + 1) * HH], 0.0)
                return carry2

            lax.fori_loop(0, blk // 8, oct_body, 0, unroll=2)
            sds = [pltpu.async_copy(ub.at[pl.ds(j * CH_E, CH_E)],
                                    acc.at[idxd.at[j]], ssem, add=True)
                   for j in range(KCH_E)]
            for d in sds:
                d.wait()

        @pl.when(b % 2 == 0)
        def _():
            run(idxs0, idxd0, idxs1, idxd1)

        @pl.when(b % 2 == 1)
        def _():
            run(idxs1, idxd1, idxs0, idxd0)

        return carry

    lax.fori_loop(0, nblk, blk_body, 0)
    plsc.subcore_barrier()
    pltpu.sync_copy(acc.at[pl.ds(s * rpt, rpt)],
                    out_hbm.at[c, pl.ds(s * rpt, rpt)])


def _make_edge(n, e):
    mesh = plsc.VectorSubcoreMesh(core_axis_name="c", subcore_axis_name="s")
    blk = CH_E * KCH_E
    return functools.partial(
        pl.kernel, functools.partial(_edge_kernel, n, e),
        out_type=jax.ShapeDtypeStruct((NC, n, HH), F32),
        mesh=mesh,
        compiler_params=pltpu.CompilerParams(use_tc_tiling_on_sc=False),
        scratch_types=[
            pltpu.VMEM_SHARED((n, HH), F32),
            pltpu.VMEM((KCH_E, CH_E), jnp.int32),
            pltpu.VMEM((KCH_E, CH_E), jnp.int32),
            pltpu.VMEM((KCH_E, CH_E), jnp.int32),
            pltpu.VMEM((KCH_E, CH_E), jnp.int32),
            pltpu.VMEM((blk, HH), F32),
            pltpu.VMEM((blk, HH), F32),
            pltpu.VMEM((blk // 8, 128), F32),
            pltpu.SemaphoreType.DMA,
            pltpu.SemaphoreType.DMA,
            pltpu.SemaphoreType.DMA,
            pltpu.SemaphoreType.DMA,
            pltpu.SemaphoreType.DMA,
        ])()


_RB = 2000


def _row_spec(tail, rb=_RB):
    nt = len(tail)
    return pl.BlockSpec((rb,) + tail, lambda i, _nt=nt: (i,) + (0,) * _nt)


def _tblw_spec():
    return pl.BlockSpec((_RB, HH), lambda i: (i, 0))


def _pair_spec():
    return pl.BlockSpec((NC, _RB, HH), lambda i: (0, i, 0))


def _full_spec(shape):
    nd = len(shape)
    return pl.BlockSpec(shape, lambda i, _nd=nd: (0,) * _nd)


def _unsplit(ref):
    return jnp.concatenate([ref[0], ref[1]], axis=-1)


def _pack(half):
    return half


def _tc1_body(dref, cref, degref, Wcb, bc3, Wg0, bg0, Wgcn1,
              xc_o, dinv_o, hd1a_o, hd1b_o):
    act = lambda z: jnp.maximum(z, 0.0)
    xd = dref[:, :20]
    xc = act(jnp.dot(cref[...], Wcb[...], preferred_element_type=F32) + bc3[...])
    xg = act(jnp.dot(jnp.concatenate([xd, xc], axis=-1), Wg0[...],
                     preferred_element_type=F32) + bg0[...])
    cnt = degref[0][:, 0:1] + degref[1][:, 0:1]
    dinv = lax.rsqrt(cnt + 1.0)
    hd1 = jnp.dot(xg, Wgcn1[...], preferred_element_type=F32) * dinv
    xc_o[...] = xc
    dinv_o[...] = jnp.broadcast_to(dinv, dinv_o.shape)
    hd1a_o[...] = _pack(hd1[:, :HH])
    hd1b_o[...] = _pack(hd1[:, HH:])


def _tc2_body(P1, hd1a, hd1b, dinv16, Wgcn2, bgcn1, xg0_o, hd2a_o, hd2b_o):
    act = lambda z: jnp.maximum(z, 0.0)
    hd1 = jnp.concatenate([hd1a[...], hd1b[...]], axis=-1)
    P = _unsplit(P1)
    dinv = dinv16[:, 0:1]
    xg0 = act((P + hd1) * dinv + bgcn1[...])
    hd2 = jnp.dot(xg0, Wgcn2[...], preferred_element_type=F32) * dinv
    xg0_o[...] = xg0
    hd2a_o[...] = _pack(hd2[:, :HH])
    hd2b_o[...] = _pack(hd2[:, HH:])


def _tc3_body(P2, hd2a, hd2b, dinv16, xg0, dref, xc, bgcn2, Wr1, br1, Wr2, br2,
              Wf, bf, Ws0, bs0, WeS, WeD,
              hci_o, xsi_o, xsia_o, xsib_o, ua_o, ub_o, va_o, vb_o):
    act = lambda z: jnp.maximum(z, 0.0)
    hd2 = jnp.concatenate([hd2a[...], hd2b[...]], axis=-1)
    P = _unsplit(P2)
    dinv = dinv16[:, 0:1]
    xg1 = act((P + hd2) * dinv + bgcn2[...])
    x = jnp.concatenate([dref[:, :20], xc[...], xg0[...] + xg1], axis=-1)
    xd1 = act(jnp.dot(x, Wr1[...], preferred_element_type=F32) + br1[...]) + x
    xd2 = act(jnp.dot(xd1, Wr2[...], preferred_element_type=F32) + br2[...]) + xd1
    hci = act(jnp.dot(xd2, Wf[...], preferred_element_type=F32) + bf[...])
    xsi = act(jnp.dot(x, Ws0[...], preferred_element_type=F32) + bs0[...])
    u = jnp.dot(xsi, WeS[...], preferred_element_type=F32)
    v = jnp.dot(xsi, WeD[...], preferred_element_type=F32)
    hci_o[...] = hci
    xsi_o[...] = xsi
    xsia_o[...] = _pack(xsi[:, :HH])
    xsib_o[...] = _pack(xsi[:, HH:])
    ua_o[...] = _pack(u[:, :HH])
    ub_o[...] = _pack(u[:, HH:])
    va_o[...] = _pack(v[:, :HH])
    vb_o[...] = _pack(v[:, HH:])


def _tcw_body(attr, WeaT, beT, w0_o, w1_o):
    w = jnp.dot(attr[...], WeaT[...], preferred_element_type=F32) + beT[...]
    m = w.shape[0] // 8
    w0_o[...] = jnp.concatenate(
        [w[kk * m:(kk + 1) * m, :HH] for kk in range(8)], axis=1)
    w1_o[...] = jnp.concatenate(
        [w[kk * m:(kk + 1) * m, HH:] for kk in range(8)], axis=1)


def _tc4_body(xsi, Q1, ses, Wroot1, Wm1t, Wm1b, bel1, xsi0_o, xsi0a_o, xsi0b_o):
    act = lambda z: jnp.maximum(z, 0.0)
    Q = _unsplit(Q1)
    se = _unsplit(ses)
    xsi0 = act(jnp.dot(xsi[...], Wroot1[...], preferred_element_type=F32)
               + jnp.dot(Q, Wm1t[...], preferred_element_type=F32)
               + jnp.dot(se, Wm1b[...], preferred_element_type=F32) + bel1[...])
    xsi0_o[...] = xsi0
    xsi0a_o[...] = _pack(xsi0[:, :HH])
    xsi0b_o[...] = _pack(xsi0[:, HH:])


def _tc5_body(xsi0, Q2, ses, hci, tref, Wroot2, Wm2t, Wm2b, bel2,
              WT, bT, Wa0, ba0, Wa1, ba1, Wy0h, by0h, Wy0o, by0o,
              Wy1h, by1h, Wy1o, by1o,
              py_o, pcf_o, py0_o, py1_o, pT_o, hsi_o):
    act = lambda z: jnp.maximum(z, 0.0)
    Q = _unsplit(Q2)
    se = _unsplit(ses)
    xsi1 = act(jnp.dot(xsi0[...], Wroot2[...], preferred_element_type=F32)
               + jnp.dot(Q, Wm2t[...], preferred_element_type=F32)
               + jnp.dot(se, Wm2b[...], preferred_element_type=F32) + bel2[...])
    hsi = xsi0[...] + xsi1
    hci_v = hci[...]
    pT = jax.nn.sigmoid(jnp.dot(hsi, WT[...], preferred_element_type=F32) + bT[...])
    h = jnp.concatenate([hci_v, hsi], axis=-1)
    a0 = jax.nn.softmax(jnp.dot(h, Wa0[...], preferred_element_type=F32) + ba0[...], axis=-1)
    y0 = a0[:, :H] * hci_v + a0[:, H:] * hsi
    a1 = jax.nn.softmax(jnp.dot(h, Wa1[...], preferred_element_type=F32) + ba1[...], axis=-1)
    y1 = a1[:, :H] * hci_v + a1[:, H:] * hsi
    y0 = jax.nn.sigmoid(
        jnp.dot(act(jnp.dot(y0, Wy0h[...], preferred_element_type=F32) + by0h[...]),
                Wy0o[...], preferred_element_type=F32) + by0o[...])
    y1 = jax.nn.sigmoid(
        jnp.dot(act(jnp.dot(y1, Wy1h[...], preferred_element_type=F32) + by1h[...]),
                Wy1o[...], preferred_element_type=F32) + by1o[...])
    t = tref[...]
    py_o[...] = (1.0 - t) * y0 + t * y1
    pcf_o[...] = t * y0 + (1.0 - t) * y1
    py0_o[...] = y0
    py1_o[...] = y1
    pT_o[...] = pT
    hsi_o[...] = hsi


def _tc_call(body, grid, ins, in_specs, outs, out_specs):
    out_shapes = [jax.ShapeDtypeStruct(s, F32) for s in outs]
    return pl.pallas_call(
        body,
        grid=(grid,),
        in_specs=in_specs,
        out_specs=out_specs[0] if len(outs) == 1 else out_specs,
        out_shape=out_shapes[0] if len(outs) == 1 else out_shapes,
    )(*ins)


def kernel(discrete_x, continous_x, edge_attr, t, params, edge_index):
    p = params
    n = discrete_x.shape[0]
    e = edge_index.shape[1]
    assert n % _RB == 0 and n % NS == 0 and (n // NS * HH) % 8 == 0
    assert e % (CH * KCH_A) == 0 and e % (CH * KCH_D) == 0
    assert e % (CH_E * KCH_E) == 0 and e % 8000 == 0

    src_r = edge_index[0].reshape(e // CH, CH)
    dst_r = edge_index[1].reshape(e // CH, CH)

    zeros_sc = jnp.zeros((n, HH), F32)
    ones_sc = jnp.ones((CH, HH), F32)

    Wc = p['Wc']
    Wcb = jnp.zeros((48, 24), F32)
    Wcb = Wcb.at[0:16, 0:8].set(Wc).at[16:32, 8:16].set(Wc).at[32:48, 16:24].set(Wc)
    bc3 = jnp.tile(p['bc'], 3).reshape(1, 24)
    r2 = lambda w: w.reshape(1, -1)
    We = p['We']

    deg_call = _make_deg(n, e)
    axt_call = _make_axt(n, e)
    edge_call = _make_edge(n, e)

    degparts = deg_call(dst_r, ones_sc, zeros_sc)

    xc, dinv16, hd1a, hd1b = _tc_call(
        _tc1_body, n // _RB,
        [discrete_x, continous_x, degparts, Wcb, bc3, p['Wg0'],
         r2(p['bg0']), p['Wgcn1']],
        [_row_spec((26,)), _row_spec((48,)), _pair_spec(),
         _full_spec((48, 24)), _full_spec((1, 24)), _full_spec((44, H)),
         _full_spec((1, H)), _full_spec((H, H))],
        [(n, 24), (n, HH), (n, HH), (n, HH)],
        [_row_spec((24,)), _row_spec((HH,)), _tblw_spec(), _tblw_spec()])

    P1 = axt_call(src_r, dst_r, hd1a, hd1b, zeros_sc)

    xg0, hd2a, hd2b = _tc_call(
        _tc2_body, n // _RB,
        [P1, hd1a, hd1b, dinv16, p['Wgcn2'], r2(p['bgcn1'])],
        [_pair_spec(), _tblw_spec(), _tblw_spec(), _row_spec((HH,)),
         _full_spec((H, H)), _full_spec((1, H))],
        [(n, H), (n, HH), (n, HH)],
        [_row_spec((H,)), _tblw_spec(), _tblw_spec()])

    P2 = axt_call(src_r, dst_r, hd2a, hd2b, zeros_sc)

    hci, xsi, xsia, xsib, ua, ub_, va, vb_ = _tc_call(
        _tc3_body, n // _RB,
        [P2, hd2a, hd2b, dinv16, xg0, discrete_x, xc, r2(p['bgcn2']),
         p['Wr1'], r2(p['br1']), p['Wr2'], r2(p['br2']),
         p['Wf'], r2(p['bf']), p['Ws0'], r2(p['bs0']),
         We[:H], We[H:2 * H]],
        [_pair_spec(), _tblw_spec(), _tblw_spec(), _row_spec((HH,)),
         _row_spec((H,)), _row_spec((26,)), _row_spec((24,)),
         _full_spec((1, H)),
         _full_spec((76, 76)), _full_spec((1, 76)),
         _full_spec((76, 76)), _full_spec((1, 76)),
         _full_spec((76, H)), _full_spec((1, H)),
         _full_spec((76, H)), _full_spec((1, H)),
         _full_spec((H, H)), _full_spec((H, H))],
        [(n, H), (n, H)] + [(n, HH)] * 6,
        [_row_spec((H,)), _row_spec((H,))] + [_tblw_spec()] * 6)

    w2a, w2b = _tc_call(
        _tcw_body, e // 8000,
        [edge_attr, We[2 * H:], r2(p['be'])],
        [pl.BlockSpec((8000, 4), lambda i: (i, 0)),
         _full_spec((4, H)), _full_spec((1, H))],
        [(e // 8, 128), (e // 8, 128)],
        [pl.BlockSpec((1000, 128), lambda i: (i, 0)),
         pl.BlockSpec((1000, 128), lambda i: (i, 0))])

    perm = lambda ix: jnp.transpose(
        ix.reshape(e // 8000, 8, 1000), (0, 2, 1)).reshape(e // CH_E, CH_E)
    ses = edge_call(perm(edge_index[0]), perm(edge_index[1]), w2a, w2b,
                    ua, ub_, va, vb_, zeros_sc)
    Q1 = axt_call(src_r, dst_r, xsia, xsib, zeros_sc)

    xsi0, xsi0a, xsi0b = _tc_call(
        _tc4_body, n // _RB,
        [xsi, Q1, ses, p['Wroot1'], p['Wmsg1'][:H], p['Wmsg1'][H:],
         r2(p['bel1'])],
        [_row_spec((H,)), _pair_spec(), _pair_spec(),
         _full_spec((H, H)), _full_spec((H, H)), _full_spec((H, H)),
         _full_spec((1, H))],
        [(n, H), (n, HH), (n, HH)],
        [_row_spec((H,)), _tblw_spec(), _tblw_spec()])

    Q2 = axt_call(src_r, dst_r, xsi0a, xsi0b, zeros_sc)

    py, pcf, py0, py1, pT, hsi = _tc_call(
        _tc5_body, n // _RB,
        [xsi0, Q2, ses, hci, t, p['Wroot2'], p['Wmsg2'][:H],
         p['Wmsg2'][H:], r2(p['bel2']), p['WT'], r2(p['bT']), p['Wa0'],
         r2(p['ba0']), p['Wa1'], r2(p['ba1']), p['Wy0h'], r2(p['by0h']),
         p['Wy0o'], r2(p['by0o']), p['Wy1h'], r2(p['by1h']), p['Wy1o'],
         r2(p['by1o'])],
        [_row_spec((H,)), _pair_spec(), _pair_spec(), _row_spec((H,)),
         _row_spec((1,)),
         _full_spec((H, H)), _full_spec((H, H)), _full_spec((H, H)),
         _full_spec((1, H)), _full_spec((H, 1)), _full_spec((1, 1)),
         _full_spec((2 * H, 2 * H)), _full_spec((1, 2 * H)),
         _full_spec((2 * H, 2 * H)), _full_spec((1, 2 * H)),
         _full_spec((H, H)), _full_spec((1, H)), _full_spec((H, 1)),
         _full_spec((1, 1)),
         _full_spec((H, H)), _full_spec((1, H)), _full_spec((H, 1)),
         _full_spec((1, 1))],
        [(n, 1)] * 5 + [(n, H)],
        [_row_spec((1,))] * 5 + [_row_spec((H,))])

    return (py, pcf, py0, py1, pT, hci, hsi)


# speedup vs baseline: 1.0261x; 1.0261x over previous
import functools

import jax
import jax.numpy as jnp
from jax import lax
from jax.experimental import pallas as pl
from jax.experimental.pallas import tpu as pltpu
from jax.experimental.pallas import tpu_sc as plsc

F32 = jnp.float32
H = 32
HH = 16
NC = 2
NS = 16
CH = 128
KCH_A = 10
KCH_D = 25
KCH_E = 4
CH_E = 128


def _axt_kernel(n, e, src_hbm, dst_hbm, tbl0, tbl1, zeros_hbm, out_hbm,
                acc, idxs0, idxd0, idxs1, idxd1, rows, isem, gsem, ssem):
    c = lax.axis_index("c")
    s = lax.axis_index("s")
    rpt = n // NS
    nbt = e // CH // KCH_A
    nblk = (nbt - s + NS - 1) // NS
    pltpu.sync_copy(zeros_hbm.at[pl.ds(s * rpt, rpt)], acc.at[pl.ds(s * rpt, rpt)])
    pltpu.async_copy(src_hbm.at[pl.ds(s * KCH_A, KCH_A)], idxs0, isem)
    pltpu.async_copy(dst_hbm.at[pl.ds(s * KCH_A, KCH_A)], idxd0, isem)
    plsc.subcore_barrier()

    def blk_body(b, carry):
        def run(idxs, idxd, oidxs, oidxd):
            pltpu.make_async_copy(src_hbm.at[pl.ds(0, KCH_A)], idxs, isem).wait()
            pltpu.make_async_copy(dst_hbm.at[pl.ds(0, KCH_A)], idxd, isem).wait()

            @pl.when(b + 1 < nblk)
            def _():
                row1 = (s + (b + 1) * NS) * KCH_A
                pltpu.async_copy(src_hbm.at[pl.ds(row1, KCH_A)], oidxs, isem)
                pltpu.async_copy(dst_hbm.at[pl.ds(row1, KCH_A)], oidxd, isem)

            @pl.when(c == 0)
            def _():
                gds = [pltpu.async_copy(tbl0.at[idxs.at[j]], rows.at[j], gsem)
                       for j in range(KCH_A)]
                for d in gds:
                    d.wait()

            @pl.when(c == 1)
            def _():
                gds = [pltpu.async_copy(tbl1.at[idxs.at[j]], rows.at[j], gsem)
                       for j in range(KCH_A)]
                for d in gds:
                    d.wait()

            sds = [pltpu.async_copy(rows.at[j], acc.at[idxd.at[j]], ssem,
                                    add=True)
                   for j in range(KCH_A)]
            for d in sds:
                d.wait()

        @pl.when(b % 2 == 0)
        def _():
            run(idxs0, idxd0, idxs1, idxd1)

        @pl.when(b % 2 == 1)
        def _():
            run(idxs1, idxd1, idxs0, idxd0)

        return carry

    lax.fori_loop(0, nblk, blk_body, 0)
    plsc.subcore_barrier()
    pltpu.sync_copy(acc.at[pl.ds(s * rpt, rpt)],
                    out_hbm.at[c, pl.ds(s * rpt, rpt)])


def _make_axt(n, e):
    mesh = plsc.VectorSubcoreMesh(core_axis_name="c", subcore_axis_name="s")
    return functools.partial(
        pl.kernel, functools.partial(_axt_kernel, n, e),
        out_type=jax.ShapeDtypeStruct((NC, n, HH), F32),
        mesh=mesh,
        compiler_params=pltpu.CompilerParams(use_tc_tiling_on_sc=False),
        scratch_types=[
            pltpu.VMEM_SHARED((n, HH), F32),
            pltpu.VMEM((KCH_A, CH), jnp.int32),
            pltpu.VMEM((KCH_A, CH), jnp.int32),
            pltpu.VMEM((KCH_A, CH), jnp.int32),
            pltpu.VMEM((KCH_A, CH), jnp.int32),
            pltpu.VMEM((KCH_A, CH, HH), F32),
            pltpu.SemaphoreType.DMA,
            pltpu.SemaphoreType.DMA,
            pltpu.SemaphoreType.DMA,
        ])()


def _deg_kernel(n, e, dst_hbm, ones_hbm, zeros_hbm, out_hbm,
                acc, idxd, ones_v, ssem):
    c = lax.axis_index("c")
    s = lax.axis_index("s")
    wid = c * NS + s
    rpt = n // NS
    nbt = e // CH // KCH_D
    nblk = (nbt - wid + NC * NS - 1) // (NC * NS)
    pltpu.sync_copy(zeros_hbm.at[pl.ds(s * rpt, rpt)], acc.at[pl.ds(s * rpt, rpt)])
    pltpu.sync_copy(ones_hbm, ones_v)
    plsc.subcore_barrier()

    def blk_body(b, carry):
        row0 = (wid + b * NC * NS) * KCH_D
        pltpu.sync_copy(dst_hbm.at[pl.ds(row0, KCH_D)], idxd)
        sds = [pltpu.async_copy(ones_v, acc.at[idxd.at[j]], ssem, add=True)
               for j in range(KCH_D)]
        for d in sds:
            d.wait()
        return carry

    lax.fori_loop(0, nblk, blk_body, 0)
    plsc.subcore_barrier()
    pltpu.sync_copy(acc.at[pl.ds(s * rpt, rpt)],
                    out_hbm.at[c, pl.ds(s * rpt, rpt)])


def _make_deg(n, e):
    mesh = plsc.VectorSubcoreMesh(core_axis_name="c", subcore_axis_name="s")
    return functools.partial(
        pl.kernel, functools.partial(_deg_kernel, n, e),
        out_type=jax.ShapeDtypeStruct((NC, n, HH), F32),
        mesh=mesh,
        compiler_params=pltpu.CompilerParams(use_tc_tiling_on_sc=False),
        scratch_types=[
            pltpu.VMEM_SHARED((n, HH), F32),
            pltpu.VMEM((KCH_D, CH), jnp.int32),
            pltpu.VMEM((CH, HH), F32),
            pltpu.SemaphoreType.DMA,
        ])()


def _edge_kernel(n, e, src_hbm, dst_hbm, w0_hbm, w1_hbm, u0, u1, v0, v1,
                 zeros_hbm, out_hbm, acc, idxs0, idxd0, idxs1, idxd1,
                 ub, vb, wb, isem, wsem, gsem1, gsem2, ssem):
    c = lax.axis_index("c")
    s = lax.axis_index("s")
    rpt = n // NS
    blk = CH_E * KCH_E
    nbt = e // blk
    nblk = (nbt - s + NS - 1) // NS
    pltpu.sync_copy(zeros_hbm.at[pl.ds(s * rpt, rpt)], acc.at[pl.ds(s * rpt, rpt)])
    pltpu.async_copy(src_hbm.at[pl.ds(s * KCH_E, KCH_E)], idxs0, isem)
    pltpu.async_copy(dst_hbm.at[pl.ds(s * KCH_E, KCH_E)], idxd0, isem)
    plsc.subcore_barrier()

    def blk_body(b, carry):
        def run(idxs, idxd, oidxs, oidxd):
            g = s + b * NS
            base = g * blk
            pltpu.make_async_copy(src_hbm.at[pl.ds(0, KCH_E)], idxs, isem).wait()
            pltpu.make_async_copy(dst_hbm.at[pl.ds(0, KCH_E)], idxd, isem).wait()

            @pl.when(b + 1 < nblk)
            def _():
                row1 = (s + (b + 1) * NS) * KCH_E
                pltpu.async_copy(src_hbm.at[pl.ds(row1, KCH_E)], oidxs, isem)
                pltpu.async_copy(dst_hbm.at[pl.ds(row1, KCH_E)], oidxd, isem)

            @pl.when(c == 0)
            def _():
                wd = pltpu.async_copy(w0_hbm.at[pl.ds(base // 8, blk // 8)],
                                      wb, wsem)
                gds = [pltpu.async_copy(u0.at[idxs.at[j]],
                                        ub.at[pl.ds(j * CH_E, CH_E)], gsem1)
                       for j in range(KCH_E)]
                gds += [pltpu.async_copy(v0.at[idxd.at[j]],
                                         vb.at[pl.ds(j * CH_E, CH_E)], gsem2)
                        for j in range(KCH_E)]
                for d in gds:
                    d.wait()
                wd.wait()

            @pl.when(c == 1)
            def _():
                wd = pltpu.async_copy(w1_hbm.at[pl.ds(base // 8, blk // 8)],
                                      wb, wsem)
                gds = [pltpu.async_copy(u1.at[idxs.at[j]],
                                        ub.at[pl.ds(j * CH_E, CH_E)], gsem1)
                       for j in range(KCH_E)]
                gds += [pltpu.async_copy(v1.at[idxd.at[j]],
                                         vb.at[pl.ds(j * CH_E, CH_E)], gsem2)
                        for j in range(KCH_E)]
                for d in gds:
                    d.wait()
                wd.wait()

            def oct_body(q, carry2):
                for kk in range(8):
                    r = q * 8 + kk
                    ub[r] = jnp.maximum(
                        ub[r] + vb[r] + wb[q, kk * HH:(kk + 1) * HH], 0.0)
                return carry2

            lax.fori_loop(0, blk // 8, oct_body, 0, unroll=2)
            sds = [pltpu.async_copy(ub.at[pl.ds(j * CH_E, CH_E)],
                                    acc.at[idxd.at[j]], ssem, add=True)
                   for j in range(KCH_E)]
            for d in sds:
                d.wait()

        @pl.when(b % 2 == 0)
        def _():
            run(idxs0, idxd0, idxs1, idxd1)

        @pl.when(b % 2 == 1)
        def _():
            run(idxs1, idxd1, idxs0, idxd0)

        return carry

    lax.fori_loop(0, nblk, blk_body, 0)
    plsc.subcore_barrier()
    pltpu.sync_copy(acc.at[pl.ds(s * rpt, rpt)],
                    out_hbm.at[c, pl.ds(s * rpt, rpt)])


def _make_edge(n, e):
    mesh = plsc.VectorSubcoreMesh(core_axis_name="c", subcore_axis_name="s")
    blk = CH_E * KCH_E
    return functools.partial(
        pl.kernel, functools.partial(_edge_kernel, n, e),
        out_type=jax.ShapeDtypeStruct((NC, n, HH), F32),
        mesh=mesh,
        compiler_params=pltpu.CompilerParams(use_tc_tiling_on_sc=False),
        scratch_types=[
            pltpu.VMEM_SHARED((n, HH), F32),
            pltpu.VMEM((KCH_E, CH_E), jnp.int32),
            pltpu.VMEM((KCH_E, CH_E), jnp.int32),
            pltpu.VMEM((KCH_E, CH_E), jnp.int32),
            pltpu.VMEM((KCH_E, CH_E), jnp.int32),
            pltpu.VMEM((blk, HH), F32),
            pltpu.VMEM((blk, HH), F32),
            pltpu.VMEM((blk // 8, 128), F32),
            pltpu.SemaphoreType.DMA,
            pltpu.SemaphoreType.DMA,
            pltpu.SemaphoreType.DMA,
            pltpu.SemaphoreType.DMA,
            pltpu.SemaphoreType.DMA,
        ])()


_RB = 2000


def _row_spec(tail, rb=_RB):
    nt = len(tail)
    return pl.BlockSpec((rb,) + tail, lambda i, _nt=nt: (i,) + (0,) * _nt)


def _tblw_spec():
    return pl.BlockSpec((_RB, HH), lambda i: (i, 0))


def _pair_spec():
    return pl.BlockSpec((NC, _RB, HH), lambda i: (0, i, 0))


def _full_spec(shape):
    nd = len(shape)
    return pl.BlockSpec(shape, lambda i, _nd=nd: (0,) * _nd)


def _unsplit(ref):
    return jnp.concatenate([ref[0], ref[1]], axis=-1)


def _pack(half):
    return half


def _tc1_body(dref, cref, degref, Wcb, bc3, Wg0, bg0, Wgcn1,
              xc_o, dinv_o, hd1a_o, hd1b_o):
    act = lambda z: jnp.maximum(z, 0.0)
    xd = dref[:, :20]
    xc = act(jnp.dot(cref[...], Wcb[...], preferred_element_type=F32) + bc3[...])
    xg = act(jnp.dot(jnp.concatenate([xd, xc], axis=-1), Wg0[...],
                     preferred_element_type=F32) + bg0[...])
    cnt = degref[0][:, 0:1] + degref[1][:, 0:1]
    dinv = lax.rsqrt(cnt + 1.0)
    hd1 = jnp.dot(xg, Wgcn1[...], preferred_element_type=F32) * dinv
    xc_o[...] = xc
    dinv_o[...] = jnp.broadcast_to(dinv, dinv_o.shape)
    hd1a_o[...] = _pack(hd1[:, :HH])
    hd1b_o[...] = _pack(hd1[:, HH:])


def _tc2_body(P1, hd1a, hd1b, dinv16, Wgcn2, bgcn1, xg0_o, hd2a_o, hd2b_o):
    act = lambda z: jnp.maximum(z, 0.0)
    hd1 = jnp.concatenate([hd1a[...], hd1b[...]], axis=-1)
    P = _unsplit(P1)
    dinv = dinv16[:, 0:1]
    xg0 = act((P + hd1) * dinv + bgcn1[...])
    hd2 = jnp.dot(xg0, Wgcn2[...], preferred_element_type=F32) * dinv
    xg0_o[...] = xg0
    hd2a_o[...] = _pack(hd2[:, :HH])
    hd2b_o[...] = _pack(hd2[:, HH:])


def _tc3_body(P2, hd2a, hd2b, dinv16, xg0, dref, xc, bgcn2, Wr1, br1, Wr2, br2,
              Wf, bf, Ws0, bs0, WeS, WeD,
              hci_o, xsi_o, xsia_o, xsib_o, ua_o, ub_o, va_o, vb_o):
    act = lambda z: jnp.maximum(z, 0.0)
    hd2 = jnp.concatenate([hd2a[...], hd2b[...]], axis=-1)
    P = _unsplit(P2)
    dinv = dinv16[:, 0:1]
    xg1 = act((P + hd2) * dinv + bgcn2[...])
    x = jnp.concatenate([dref[:, :20], xc[...], xg0[...] + xg1], axis=-1)
    xd1 = act(jnp.dot(x, Wr1[...], preferred_element_type=F32) + br1[...]) + x
    xd2 = act(jnp.dot(xd1, Wr2[...], preferred_element_type=F32) + br2[...]) + xd1
    hci = act(jnp.dot(xd2, Wf[...], preferred_element_type=F32) + bf[...])
    xsi = act(jnp.dot(x, Ws0[...], preferred_element_type=F32) + bs0[...])
    u = jnp.dot(xsi, WeS[...], preferred_element_type=F32)
    v = jnp.dot(xsi, WeD[...], preferred_element_type=F32)
    hci_o[...] = hci
    xsi_o[...] = xsi
    xsia_o[...] = _pack(xsi[:, :HH])
    xsib_o[...] = _pack(xsi[:, HH:])
    ua_o[...] = _pack(u[:, :HH])
    ub_o[...] = _pack(u[:, HH:])
    va_o[...] = _pack(v[:, :HH])
    vb_o[...] = _pack(v[:, HH:])


def _tcw_body(attr, WeaT, beT, w0_o, w1_o):
    w = jnp.dot(attr[...], WeaT[...], preferred_element_type=F32) + beT[...]
    m = w.shape[0] // 8
    w0_o[...] = jnp.concatenate(
        [w[kk * m:(kk + 1) * m, :HH] for kk in range(8)], axis=1)
    w1_o[...] = jnp.concatenate(
        [w[kk * m:(kk + 1) * m, HH:] for kk in range(8)], axis=1)


def _tc4_body(xsi, Q1, ses, Wroot1, Wm1t, Wm1b, bel1, xsi0_o, xsi0a_o, xsi0b_o):
    act = lambda z: jnp.maximum(z, 0.0)
    Q = _unsplit(Q1)
    se = _unsplit(ses)
    xsi0 = act(jnp.dot(xsi[...], Wroot1[...], preferred_element_type=F32)
               + jnp.dot(Q, Wm1t[...], preferred_element_type=F32)
               + jnp.dot(se, Wm1b[...], preferred_element_type=F32) + bel1[...])
    xsi0_o[...] = xsi0
    xsi0a_o[...] = _pack(xsi0[:, :HH])
    xsi0b_o[...] = _pack(xsi0[:, HH:])


def _tc5_body(xsi0, Q2, ses, hci, tref, Wroot2, Wm2t, Wm2b, bel2,
              WT, bT, Wa0, ba0, Wa1, ba1, Wy0h, by0h, Wy0o, by0o,
              Wy1h, by1h, Wy1o, by1o,
              py_o, pcf_o, py0_o, py1_o, pT_o, hsi_o):
    act = lambda z: jnp.maximum(z, 0.0)
    Q = _unsplit(Q2)
    se = _unsplit(ses)
    xsi1 = act(jnp.dot(xsi0[...], Wroot2[...], preferred_element_type=F32)
               + jnp.dot(Q, Wm2t[...], preferred_element_type=F32)
               + jnp.dot(se, Wm2b[...], preferred_element_type=F32) + bel2[...])
    hsi = xsi0[...] + xsi1
    hci_v = hci[...]
    pT = jax.nn.sigmoid(jnp.dot(hsi, WT[...], preferred_element_type=F32) + bT[...])
    h = jnp.concatenate([hci_v, hsi], axis=-1)
    a0 = jax.nn.softmax(jnp.dot(h, Wa0[...], preferred_element_type=F32) + ba0[...], axis=-1)
    y0 = a0[:, :H] * hci_v + a0[:, H:] * hsi
    a1 = jax.nn.softmax(jnp.dot(h, Wa1[...], preferred_element_type=F32) + ba1[...], axis=-1)
    y1 = a1[:, :H] * hci_v + a1[:, H:] * hsi
    y0 = jax.nn.sigmoid(
        jnp.dot(act(jnp.dot(y0, Wy0h[...], preferred_element_type=F32) + by0h[...]),
                Wy0o[...], preferred_element_type=F32) + by0o[...])
    y1 = jax.nn.sigmoid(
        jnp.dot(act(jnp.dot(y1, Wy1h[...], preferred_element_type=F32) + by1h[...]),
                Wy1o[...], preferred_element_type=F32) + by1o[...])
    t = tref[...]
    py_o[...] = (1.0 - t) * y0 + t * y1
    pcf_o[...] = t * y0 + (1.0 - t) * y1
    py0_o[...] = y0
    py1_o[...] = y1
    pT_o[...] = pT
    hsi_o[...] = hsi


def _tc_call(body, grid, ins, in_specs, outs, out_specs):
    out_shapes = [jax.ShapeDtypeStruct(s, F32) for s in outs]
    return pl.pallas_call(
        body,
        grid=(grid,),
        in_specs=in_specs,
        out_specs=out_specs[0] if len(outs) == 1 else out_specs,
        out_shape=out_shapes[0] if len(outs) == 1 else out_shapes,
    )(*ins)


def kernel(discrete_x, continous_x, edge_attr, t, params, edge_index):
    p = params
    n = discrete_x.shape[0]
    e = edge_index.shape[1]
    assert n % _RB == 0 and n % NS == 0 and (n // NS * HH) % 8 == 0
    assert e % (CH * KCH_A) == 0 and e % (CH * KCH_D) == 0
    assert e % (CH_E * KCH_E) == 0 and e % 8000 == 0

    src_r = edge_index[0].reshape(e // CH, CH)
    dst_r = edge_index[1].reshape(e // CH, CH)

    zeros_sc = jnp.zeros((n, HH), F32)
    ones_sc = jnp.ones((CH, HH), F32)

    Wc = p['Wc']
    Wcb = jnp.zeros((48, 24), F32)
    Wcb = Wcb.at[0:16, 0:8].set(Wc).at[16:32, 8:16].set(Wc).at[32:48, 16:24].set(Wc)
    bc3 = jnp.tile(p['bc'], 3).reshape(1, 24)
    r2 = lambda w: w.reshape(1, -1)
    We = p['We']

    deg_call = _make_deg(n, e)
    axt_call = _make_axt(n, e)
    edge_call = _make_edge(n, e)

    degparts = deg_call(dst_r, ones_sc, zeros_sc)

    xc, dinv16, hd1a, hd1b = _tc_call(
        _tc1_body, n // _RB,
        [discrete_x, continous_x, degparts, Wcb, bc3, p['Wg0'],
         r2(p['bg0']), p['Wgcn1']],
        [_row_spec((26,)), _row_spec((48,)), _pair_spec(),
         _full_spec((48, 24)), _full_spec((1, 24)), _full_spec((44, H)),
         _full_spec((1, H)), _full_spec((H, H))],
        [(n, 24), (n, HH), (n, HH), (n, HH)],
        [_row_spec((24,)), _row_spec((HH,)), _tblw_spec(), _tblw_spec()])

    P1 = axt_call(src_r, dst_r, hd1a, hd1b, zeros_sc)

    xg0, hd2a, hd2b = _tc_call(
        _tc2_body, n // _RB,
        [P1, hd1a, hd1b, dinv16, p['Wgcn2'], r2(p['bgcn1'])],
        [_pair_spec(), _tblw_spec(), _tblw_spec(), _row_spec((HH,)),
         _full_spec((H, H)), _full_spec((1, H))],
        [(n, H), (n, HH), (n, HH)],
        [_row_spec((H,)), _tblw_spec(), _tblw_spec()])

    P2 = axt_call(src_r, dst_r, hd2a, hd2b, zeros_sc)

    hci, xsi, xsia, xsib, ua, ub_, va, vb_ = _tc_call(
        _tc3_body, n // _RB,
        [P2, hd2a, hd2b, dinv16, xg0, discrete_x, xc, r2(p['bgcn2']),
         p['Wr1'], r2(p['br1']), p['Wr2'], r2(p['br2']),
         p['Wf'], r2(p['bf']), p['Ws0'], r2(p['bs0']),
         We[:H], We[H:2 * H]],
        [_pair_spec(), _tblw_spec(), _tblw_spec(), _row_spec((HH,)),
         _row_spec((H,)), _row_spec((26,)), _row_spec((24,)),
         _full_spec((1, H)),
         _full_spec((76, 76)), _full_spec((1, 76)),
         _full_spec((76, 76)), _full_spec((1, 76)),
         _full_spec((76, H)), _full_spec((1, H)),
         _full_spec((76, H)), _full_spec((1, H)),
         _full_spec((H, H)), _full_spec((H, H))],
        [(n, H), (n, H)] + [(n, HH)] * 6,
        [_row_spec((H,)), _row_spec((H,))] + [_tblw_spec()] * 6)

    w2a, w2b = _tc_call(
        _tcw_body, e // 8000,
        [edge_attr, We[2 * H:], r2(p['be'])],
        [pl.BlockSpec((8000, 4), lambda i: (i, 0)),
         _full_spec((4, H)), _full_spec((1, H))],
        [(e // 8, 128), (e // 8, 128)],
        [pl.BlockSpec((1000, 128), lambda i: (i, 0)),
         pl.BlockSpec((1000, 128), lambda i: (i, 0))])

    perm = lambda ix: jnp.transpose(
        ix.reshape(e // 8000, 8, 1000), (0, 2, 1)).reshape(e // CH_E, CH_E)
    ses = edge_call(perm(edge_index[0]), perm(edge_index[1]), w2a, w2b,
                    ua, ub_, va, vb_, zeros_sc)
    Q1 = axt_call(src_r, dst_r, xsia, xsib, zeros_sc)

    xsi0, xsi0a, xsi0b = _tc_call(
        _tc4_body, n // _RB,
        [xsi, Q1, ses, p['Wroot1'], p['Wmsg1'][:H], p['Wmsg1'][H:],
         r2(p['bel1'])],
        [_row_spec((H,)), _pair_spec(), _pair_spec(),
         _full_spec((H, H)), _full_spec((H, H)), _full_spec((H, H)),
         _full_spec((1, H))],
        [(n, H), (n, HH), (n, HH)],
        [_row_spec((H,)), _tblw_spec(), _tblw_spec()])

    Q2 = axt_call(src_r, dst_r, xsi0a, xsi0b, zeros_sc)

    py, pcf, py0, py1, pT, hsi = _tc_call(
        _tc5_body, n // _RB,
        [xsi0, Q2, ses, hci, t, p['Wroot2'], p['Wmsg2'][:H],
         p['Wmsg2'][H:], r2(p['bel2']), p['WT'], r2(p['bT']), p['Wa0'],
         r2(p['ba0']), p['Wa1'], r2(p['ba1']), p['Wy0h'], r2(p['by0h']),
         p['Wy0o'], r2(p['by0o']), p['Wy1h'], r2(p['by1h']), p['Wy1o'],
         r2(p['by1o'])],
        [_row_spec((H,)), _pair_spec(), _pair_spec(), _row_spec((H,)),
         _row_spec((1,)),
         _full_spec((H, H)), _full_spec((H, H)), _full_spec((H, H)),
         _full_spec((1, H)), _full_spec((H, 1)), _full_spec((1, 1)),
         _full_spec((2 * H, 2 * H)), _full_spec((1, 2 * H)),
         _full_spec((2 * H, 2 * H)), _full_spec((1, 2 * H)),
         _full_spec((H, H)), _full_spec((1, H)), _full_spec((H, 1)),
         _full_spec((1, 1)),
         _full_spec((H, H)), _full_spec((1, H)), _full_spec((H, 1)),
         _full_spec((1, 1))],
        [(n, 1)] * 5 + [(n, H)],
        [_row_spec((1,))] * 5 + [_row_spec((H,))])

    return (py, pcf, py0, py1, pT, hci, hsi)
